# Initial kernel scaffold; baseline (speedup 1.0000x reference)
import jax
import jax.numpy as jnp
from jax.experimental import pallas as pl


def kernel(X_0, edge_index, A_values, U, W, Omega_1, Omega_2, bias, fw_mitr, bw_mitr):
    raise NotImplementedError("write your pallas kernel here")


# speedup vs baseline: 3.2044x; 3.2044x over previous
import functools

import jax
import jax.numpy as jnp
from jax import lax
from jax.experimental import pallas as pl
from jax.experimental.pallas import tpu as pltpu
from jax.experimental.pallas import tpu_sc as plsc

_KAPPA = 0.99
_NC = 2
_NS = 16
_L = 16


def _project_linf(W, v):
    a_abs = jnp.abs(W)
    row_l1 = a_abs.sum(axis=1, keepdims=True)
    u = jnp.sort(a_abs, axis=1)[:, ::-1]
    css = jnp.cumsum(u, axis=1)
    j = jnp.arange(1, W.shape[1] + 1, dtype=W.dtype)
    rho = jnp.sum((u * j) > (css - v), axis=1)
    rho = jnp.maximum(rho, 1)
    theta = (jnp.take_along_axis(css, (rho - 1)[:, None], axis=1) - v) / rho[
        :, None
    ].astype(W.dtype)
    W_proj = jnp.sign(W) * jnp.maximum(a_abs - theta, 0.0)
    return jnp.where(row_l1 > v, W_proj, W)


@functools.lru_cache(maxsize=None)
def _make_mm(NP, M, R, interpret=False):
    def body(x_ref, m_ref, y_ref):
        y_ref[...] = jnp.dot(x_ref[...], m_ref[...],
                             preferred_element_type=jnp.float32)

    return pl.pallas_call(
        body,
        grid=(NP // R,),
        in_specs=[
            pl.BlockSpec((R, M), lambda i: (i, 0)),
            pl.BlockSpec((M, M), lambda i: (0, 0)),
        ],
        out_specs=pl.BlockSpec((R, M), lambda i: (i, 0)),
        out_shape=jax.ShapeDtypeStruct((NP, M), jnp.float32),
        interpret=interpret,
    )


@functools.lru_cache(maxsize=None)
def _make_fused(NP, M, R, interpret=False):
    def body(s_ref, b_ref, w_ref, x_ref, y_ref):
        x = s_ref[0] + s_ref[1] + b_ref[0] + b_ref[1]
        x = jnp.maximum(x, 0.0)
        x_ref[...] = x
        y_ref[...] = jnp.dot(x, w_ref[...],
                             preferred_element_type=jnp.float32)

    return pl.pallas_call(
        body,
        grid=(NP // R,),
        in_specs=[
            pl.BlockSpec((_NC, R, M), lambda i: (0, i, 0)),
            pl.BlockSpec((_NC, R, M), lambda i: (0, i, 0)),
            pl.BlockSpec((M, M), lambda i: (0, 0)),
        ],
        out_specs=[
            pl.BlockSpec((R, M), lambda i: (i, 0)),
            pl.BlockSpec((R, M), lambda i: (i, 0)),
        ],
        out_shape=[
            jax.ShapeDtypeStruct((NP, M), jnp.float32),
            jax.ShapeDtypeStruct((NP, M), jnp.float32),
        ],
        interpret=interpret,
    )


@functools.lru_cache(maxsize=None)
def _make_spmm(E, NP, M):
    NW = _NC * _NS
    EPT = E // NW
    K = _L
    NB = EPT // K
    RPT = NP // _NS
    HM = M // _L

    mesh = plsc.VectorSubcoreMesh(core_axis_name="c", subcore_axis_name="s")

    @functools.partial(
        pl.kernel,
        mesh=mesh,
        out_type=jax.ShapeDtypeStruct((_NC, NP, M), jnp.float32),
        scratch_types=[
            pltpu.VMEM((EPT,), jnp.int32),
            pltpu.VMEM((EPT,), jnp.int32),
            pltpu.VMEM((EPT,), jnp.float32),
            pltpu.VMEM((K, M), jnp.float32),
            pltpu.VMEM((K, M), jnp.float32),
            pltpu.VMEM_SHARED((NP, M), jnp.float32),
            pltpu.SemaphoreType.DMA,
        ],
    )
    def spmm(src_hbm, dst_hbm, vals_hbm, y_hbm, out_hbm,
             src_v, dst_v, vals_v, gbuf, sbuf, acc, sem):
        cid = lax.axis_index("c")
        sid = lax.axis_index("s")
        gid = cid * _NS + sid

        for j in range(K):
            for h in range(HM):
                sbuf[j, pl.ds(h * _L, _L)] = jnp.zeros((_L,), jnp.float32)

        def zbody(i, carry):
            pltpu.sync_copy(sbuf, acc.at[pl.ds(sid * RPT + i * K, K)])
            return carry

        lax.fori_loop(0, RPT // K, zbody, 0)

        base = gid * EPT
        pltpu.sync_copy(src_hbm.at[pl.ds(base, EPT)], src_v)
        pltpu.sync_copy(dst_hbm.at[pl.ds(base, EPT)], dst_v)
        pltpu.sync_copy(vals_hbm.at[pl.ds(base, EPT)], vals_v)
        plsc.subcore_barrier()

        def body(b, carry):
            e0 = b * K
            src16 = src_v[pl.ds(e0, K)]
            pltpu.async_copy(y_hbm.at[src16], gbuf, sem).wait()
            vals16 = vals_v[pl.ds(e0, K)]
            for j in range(K):
                vj = jnp.squeeze(lax.slice(vals16, (j,), (j + 1,)))
                for h in range(HM):
                    sl = pl.ds(h * _L, _L)
                    sbuf[j, sl] = gbuf[j, sl] * vj
            dst16 = dst_v[pl.ds(e0, K)]
            pltpu.sync_copy(sbuf, acc.at[dst16], add=True)
            return carry

        lax.fori_loop(0, NB, body, 0)
        plsc.subcore_barrier()

        pltpu.sync_copy(acc.at[pl.ds(sid * RPT, RPT)],
                        out_hbm.at[cid, pl.ds(sid * RPT, RPT)])

    return spmm


def kernel(X_0, edge_index, A_values, U, W, Omega_1, Omega_2, bias,
           fw_mitr=30, bw_mitr=30):
    M, N = X_0.shape
    E = A_values.shape[0]
    NP = ((N + _NS * _L - 1) // (_NS * _L)) * (_NS * _L)
    R = NP // 10

    Wp_t = _project_linf(W, _KAPPA).T
    O1_t = Omega_1.T
    src = edge_index[0]
    dst = edge_index[1]

    pad = jnp.zeros((NP - N, M), jnp.float32)
    Xt = jnp.concatenate([X_0.T, pad], axis=0)
    Ut = jnp.concatenate([U.T, pad], axis=0)

    mm = _make_mm(NP, M, R)
    fused = _make_fused(NP, M, R)
    spmm = _make_spmm(E, NP, M)

    Y0 = mm(Xt, Wp_t)
    S1t = mm(Ut, O1_t)
    Bt = spmm(src, dst, A_values, S1t)

    def body(_, carry):
        X, Y = carry
        St = spmm(src, dst, A_values, Y)
        X2, Y2 = fused(St, Bt, Wp_t)
        return (X2, Y2)

    X_fin, _ = lax.fori_loop(0, fw_mitr, body, (Xt, Y0))
    return X_fin[:N].T


# speedup vs baseline: 3.8163x; 1.1909x over previous
import functools

import jax
import jax.numpy as jnp
from jax import lax
from jax.experimental import pallas as pl
from jax.experimental.pallas import tpu as pltpu
from jax.experimental.pallas import tpu_sc as plsc

_KAPPA = 0.99
_NC = 2
_NS = 16
_L = 16
_K = 32


def _project_linf(W, v):
    a_abs = jnp.abs(W)
    row_l1 = a_abs.sum(axis=1, keepdims=True)
    u = jnp.sort(a_abs, axis=1)[:, ::-1]
    css = jnp.cumsum(u, axis=1)
    j = jnp.arange(1, W.shape[1] + 1, dtype=W.dtype)
    rho = jnp.sum((u * j) > (css - v), axis=1)
    rho = jnp.maximum(rho, 1)
    theta = (jnp.take_along_axis(css, (rho - 1)[:, None], axis=1) - v) / rho[
        :, None
    ].astype(W.dtype)
    W_proj = jnp.sign(W) * jnp.maximum(a_abs - theta, 0.0)
    return jnp.where(row_l1 > v, W_proj, W)


@functools.lru_cache(maxsize=None)
def _make_mm_add(NP, M, R, interpret=False):
    def body(x_ref, m_ref, a_ref, z_ref):
        z_ref[...] = a_ref[...] + jnp.dot(x_ref[...], m_ref[...],
                                          preferred_element_type=jnp.float32)

    return pl.pallas_call(
        body,
        grid=(NP // R,),
        in_specs=[
            pl.BlockSpec((R, M), lambda i: (i, 0)),
            pl.BlockSpec((M, M), lambda i: (0, 0)),
            pl.BlockSpec((R, M), lambda i: (i, 0)),
        ],
        out_specs=pl.BlockSpec((R, M), lambda i: (i, 0)),
        out_shape=jax.ShapeDtypeStruct((NP, M), jnp.float32),
        interpret=interpret,
    )


@functools.lru_cache(maxsize=None)
def _make_fused(NP, M, R, interpret=False):
    def body(s_ref, s1_ref, w_ref, x_ref, z_ref):
        x = jnp.maximum(s_ref[0] + s_ref[1], 0.0)
        x_ref[...] = x
        z_ref[...] = s1_ref[...] + jnp.dot(x, w_ref[...],
                                           preferred_element_type=jnp.float32)

    return pl.pallas_call(
        body,
        grid=(NP // R,),
        in_specs=[
            pl.BlockSpec((_NC, R, M), lambda i: (0, i, 0)),
            pl.BlockSpec((R, M), lambda i: (i, 0)),
            pl.BlockSpec((M, M), lambda i: (0, 0)),
        ],
        out_specs=[
            pl.BlockSpec((R, M), lambda i: (i, 0)),
            pl.BlockSpec((R, M), lambda i: (i, 0)),
        ],
        out_shape=[
            jax.ShapeDtypeStruct((NP, M), jnp.float32),
            jax.ShapeDtypeStruct((NP, M), jnp.float32),
        ],
        interpret=interpret,
    )


@functools.lru_cache(maxsize=None)
def _make_spmm(EP, NP, M, K, NB):
    NW = _NC * _NS
    EPT = EP // NW
    RPT = NP // _NS
    HM = M // _L
    NBUF = 2

    mesh = plsc.VectorSubcoreMesh(core_axis_name="c", subcore_axis_name="s")

    @functools.partial(
        pl.kernel,
        mesh=mesh,
        out_type=jax.ShapeDtypeStruct((_NC, NP, M), jnp.float32),
        scratch_types=[
            pltpu.VMEM((EPT,), jnp.int32),
            pltpu.VMEM((EPT,), jnp.int32),
            pltpu.VMEM((EPT,), jnp.float32),
            pltpu.VMEM((NBUF, K, M), jnp.float32),
            pltpu.VMEM((NBUF, K, M), jnp.float32),
            pltpu.VMEM_SHARED((NP, M), jnp.float32),
            pltpu.SemaphoreType.DMA,
            pltpu.SemaphoreType.DMA,
            pltpu.SemaphoreType.DMA,
            pltpu.SemaphoreType.DMA,
        ],
    )
    def spmm(src_hbm, dst_hbm, vals_hbm, z_hbm, out_hbm,
             src_v, dst_v, vals_v, gbuf, sbuf, acc,
             gsem0, gsem1, ssem0, ssem1):
        cid = lax.axis_index("c")
        sid = lax.axis_index("s")
        gid = cid * _NS + sid
        gsems = (gsem0, gsem1)
        ssems = (ssem0, ssem1)

        for j in range(K):
            for h in range(HM):
                sbuf[0, j, pl.ds(h * _L, _L)] = jnp.zeros((_L,), jnp.float32)

        def zbody(i, carry):
            pltpu.sync_copy(sbuf.at[0], acc.at[pl.ds(sid * RPT + i * K, K)])
            return carry

        lax.fori_loop(0, RPT // K, zbody, 0)

        ebase = gid * EPT
        pltpu.sync_copy(src_hbm.at[pl.ds(ebase, EPT)], src_v)
        pltpu.sync_copy(dst_hbm.at[pl.ds(ebase, EPT)], dst_v)
        pltpu.sync_copy(vals_hbm.at[pl.ds(ebase, EPT)], vals_v)
        plsc.subcore_barrier()

        def issue_gather(j, e0):
            for t in range(K // _L):
                idx = src_v[pl.ds(e0 + t * _L, _L)]
                pltpu.async_copy(z_hbm.at[idx],
                                 gbuf.at[j, pl.ds(t * _L, _L)], gsems[j])

        def wait_gather(j):
            for t in range(K // _L):
                pltpu.make_async_copy(
                    z_hbm.at[src_v[pl.ds(t * _L, _L)]],
                    gbuf.at[j, pl.ds(t * _L, _L)], gsems[j]).wait()

        def issue_scatter(j, e0):
            for t in range(K // _L):
                idx = dst_v[pl.ds(e0 + t * _L, _L)]
                pltpu.async_copy(sbuf.at[j, pl.ds(t * _L, _L)],
                                 acc.at[idx], ssems[j], add=True)

        def wait_scatter(j):
            for t in range(K // _L):
                pltpu.make_async_copy(
                    sbuf.at[j, pl.ds(t * _L, _L)],
                    acc.at[dst_v[pl.ds(t * _L, _L)]], ssems[j]).wait()

        for j in range(NBUF):
            issue_gather(j, j * K)

        def scale(j, e0):
            v16s = [vals_v[pl.ds(e0 + t * _L, _L)] for t in range(K // _L)]
            for jj in range(K):
                vj = jnp.squeeze(
                    lax.slice(v16s[jj // _L], (jj % _L,), (jj % _L + 1,)))
                for h in range(HM):
                    sl = pl.ds(h * _L, _L)
                    sbuf[j, jj, sl] = gbuf[j, jj, sl] * vj

        def body(g, carry):
            for j in range(NBUF):
                b = g * NBUF + j
                wait_gather(j)

                @pl.when(g > 0)
                def _():
                    wait_scatter(j)

                scale(j, b * K)
                issue_scatter(j, b * K)

                @pl.when(b + NBUF < NB)
                def _():
                    issue_gather(j, (b + NBUF) * K)
            return carry

        lax.fori_loop(0, NB // NBUF, body, 0)
        for j in range(NBUF):
            wait_scatter(j)

        plsc.subcore_barrier()
        pltpu.sync_copy(acc.at[pl.ds(sid * RPT, RPT)],
                        out_hbm.at[cid, pl.ds(sid * RPT, RPT)])

    return spmm


def kernel(X_0, edge_index, A_values, U, W, Omega_1, Omega_2, bias,
           fw_mitr=30, bw_mitr=30):
    M, N = X_0.shape
    E = A_values.shape[0]
    NW = _NC * _NS
    NP = ((N + _NS * _L - 1) // (_NS * _L)) * (_NS * _L)
    R = NP // 10

    GRAN = NW * _K * 8
    EP = ((E + GRAN - 1) // GRAN) * GRAN
    NB = EP // (NW * _K)

    Wp_t = _project_linf(W, _KAPPA).T
    epad = jnp.zeros((EP - E,), jnp.int32)
    src1 = jnp.concatenate([edge_index[0], epad])
    dst1 = jnp.concatenate([edge_index[1], epad])
    vals = jnp.concatenate([A_values, epad.astype(jnp.float32)])

    pad = jnp.zeros((NP - N, M), jnp.float32)
    Xt = jnp.concatenate([X_0.T, pad], axis=0)
    Ut = jnp.concatenate([U.T, pad], axis=0)
    zeros = jnp.zeros((NP, M), jnp.float32)

    mm_add = _make_mm_add(NP, M, R)
    fused = _make_fused(NP, M, R)
    spmm = _make_spmm(EP, NP, M, _K, NB)

    S1t = mm_add(Ut, Omega_1.T, zeros)
    Z0 = mm_add(Xt, Wp_t, S1t)

    def body(_, carry):
        X, Z = carry
        St = spmm(src1, dst1, vals, Z)
        X2, Z2 = fused(St, S1t, Wp_t)
        return (X2, Z2)

    X_fin, _ = lax.fori_loop(0, fw_mitr, body, (Xt, Z0))
    return X_fin[:N].T
